# baseline (speedup 1.0000x reference)
import functools

import jax
import jax.numpy as jnp
from jax import lax
from jax.experimental import pallas as pl
from jax.experimental.pallas import tpu as pltpu
from jax.experimental.pallas import tpu_sc as plsc

N_NODES = 10000
D_IN = 128
N_EDGES = 320000

NUM_CORES = 2
NUM_SUBCORES = 16
NUM_WORKERS = NUM_CORES * NUM_SUBCORES

CHUNK = 200
GATHER_BATCH = 40
NUM_BATCHES = CHUNK // GATHER_BATCH
ROWS_PER_STEP = 4
D_W = D_IN // 2


def _sc_diff_kernel(xw, src, dst, n_edges):
    mesh = plsc.VectorSubcoreMesh(
        core_axis_name="c", subcore_axis_name="s",
        num_cores=NUM_CORES, num_subcores=NUM_SUBCORES)
    epw = n_edges // NUM_WORKERS
    num_chunks = epw // CHUNK
    assert n_edges % NUM_WORKERS == 0 and epw % CHUNK == 0 and num_chunks >= 2

    @functools.partial(
        pl.kernel,
        out_type=jax.ShapeDtypeStruct((n_edges, D_W), jnp.int32),
        mesh=mesh,
        compiler_params=pltpu.CompilerParams(
            use_tc_tiling_on_sc=False, needs_layout_passes=False),
        scratch_types=[
            pltpu.VMEM((epw,), jnp.int32),
            pltpu.VMEM((epw,), jnp.int32),
            pltpu.VMEM((CHUNK, D_W), jnp.int32),
            pltpu.VMEM((CHUNK, D_W), jnp.int32),
            pltpu.VMEM((CHUNK, D_W), jnp.int32),
            pltpu.VMEM((CHUNK, D_W), jnp.int32),
            pltpu.SemaphoreType.DMA,
            pltpu.SemaphoreType.DMA,
            pltpu.SemaphoreType.DMA,
            pltpu.SemaphoreType.DMA,
        ],
    )
    def k(x_hbm, src_hbm, dst_hbm, diff_hbm,
          sidx, didx, srows0, srows1, drows0, drows1,
          sem_g0, sem_g1, sem_w0, sem_w1):
        wid = lax.axis_index("s") * NUM_CORES + lax.axis_index("c")
        base = wid * epw
        srows = (srows0, srows1)
        drows = (drows0, drows1)
        sem_g = (sem_g0, sem_g1)
        sem_w = (sem_w0, sem_w1)

        def gather_descs(cn, p):
            descs = []
            for b in range(NUM_BATCHES):
                isl = pl.ds(cn * CHUNK + b * GATHER_BATCH, GATHER_BATCH)
                rsl = pl.ds(b * GATHER_BATCH, GATHER_BATCH)
                descs.append((x_hbm.at[sidx.at[isl]], srows[p].at[rsl], sem_g[p]))
                descs.append((x_hbm.at[didx.at[isl]], drows[p].at[rsl], sem_g[p]))
            return descs

        def when(pred, fn):
            if isinstance(pred, bool):
                if pred:
                    fn()
            else:
                pl.when(pred)(fn)

        def substep(c, p):
            pp = 1 - p

            def wb_wait():
                pltpu.make_async_copy(
                    drows[pp], diff_hbm.at[pl.ds(base, CHUNK)], sem_w[pp]
                ).wait()

            def fire_next():
                for s_, d_, sm in gather_descs(c + 1, pp):
                    pltpu.async_copy(s_, d_, sm)

            when(c > 0, wb_wait)
            when(c + 1 < num_chunks, fire_next)

            for s_, d_, sm in gather_descs(c, p):
                pltpu.make_async_copy(s_, d_, sm).wait()

            def row_body(i, carry2):
                for rr in range(ROWS_PER_STEP):
                    r = i * ROWS_PER_STEP + rr
                    for kk in range(D_W // 16):
                        s = pl.ds(kk * 16, 16)
                        dv = plsc.bitcast(drows[p][r, s], jnp.bfloat16)
                        sv = plsc.bitcast(srows[p][r, s], jnp.bfloat16)
                        drows[p][r, s] = plsc.bitcast(
                            jnp.abs(dv - sv), jnp.int32)
                return carry2

            lax.fori_loop(0, CHUNK // ROWS_PER_STEP, row_body, 0)

            pltpu.async_copy(
                drows[p], diff_hbm.at[pl.ds(base + c * CHUNK, CHUNK)], sem_w[p])

        pltpu.sync_copy(src_hbm.at[pl.ds(base, epw)], sidx)
        pltpu.sync_copy(dst_hbm.at[pl.ds(base, epw)], didx)
        for s_, d_, sm in gather_descs(0, 0):
            pltpu.async_copy(s_, d_, sm)

        substep(0, 0)

        def pair_body(i, carry):
            substep(2 * i + 1, 1)
            substep(2 * i + 2, 0)
            return carry

        lax.fori_loop(0, (num_chunks - 1) // 2, pair_body, 0)
        if (num_chunks - 1) % 2 == 1:
            substep(num_chunks - 1, 1)

        last_p = (num_chunks - 1) % 2
        pltpu.make_async_copy(
            drows[last_p], diff_hbm.at[pl.ds(base, CHUNK)], sem_w[last_p]
        ).wait()

    return k(xw, src, dst)


BLOCK_E = 3200
N_SLICES = 2


def _mlp(diff_ref, w1_ref, b1_ref, w2_ref, b2_ref, out_ref):
    d = diff_ref[...]
    h = jnp.dot(d, w1_ref[...], preferred_element_type=jnp.float32)
    h = jnp.maximum(h + b1_ref[...], 0.0)
    e = jnp.dot(h.astype(jnp.bfloat16), w2_ref[...],
                preferred_element_type=jnp.float32)
    out_ref[...] = jax.nn.sigmoid(e + b2_ref[...])


def _tc_mlp_body(diff_ref, w1_ref, b1_ref, w2_ref, b2_ref, acc_ref, out_ref):
    del acc_ref
    _mlp(diff_ref, w1_ref, b1_ref, w2_ref, b2_ref, out_ref)


_WEIGHT_SPECS = [
    pl.BlockSpec((D_IN, 64), lambda i: (0, 0)),
    pl.BlockSpec((1, 64), lambda i: (0, 0)),
    pl.BlockSpec((64, D_IN), lambda i: (0, 0)),
    pl.BlockSpec((1, D_IN), lambda i: (0, 0)),
]


def _tc_mlp_slice(diff, W1, b1, W2, b2, acc, block_base):
    n_rows = diff.shape[0]
    grid = (n_rows // BLOCK_E,)
    dspec = pl.BlockSpec((BLOCK_E, D_IN), lambda i: (i, 0))
    ospec = pl.BlockSpec((BLOCK_E, D_IN), lambda i: (block_base + i, 0))
    oshape = jax.ShapeDtypeStruct((N_EDGES, D_IN), jnp.float32)
    if acc is None:
        return pl.pallas_call(
            _mlp, grid=grid,
            in_specs=[dspec] + _WEIGHT_SPECS,
            out_specs=ospec, out_shape=oshape,
        )(diff, W1, b1, W2, b2)
    return pl.pallas_call(
        _tc_mlp_body, grid=grid,
        in_specs=[dspec] + _WEIGHT_SPECS + [pl.BlockSpec(memory_space=pl.ANY)],
        out_specs=ospec, out_shape=oshape,
        input_output_aliases={5: 0},
    )(diff, W1, b1, W2, b2, acc)


def kernel(x, edge_index, W1, b1, W2, b2):
    src = edge_index[0]
    dst = edge_index[1]
    xb = x.astype(jnp.bfloat16)
    xw = jax.lax.bitcast_convert_type(
        xb.reshape(N_NODES, D_W, 2), jnp.int32)
    W1b = W1.astype(jnp.bfloat16)
    W2b = W2.astype(jnp.bfloat16)
    b1r = b1.reshape(1, 64)
    b2r = b2.reshape(1, 128)
    es = N_EDGES // N_SLICES
    diffs = [
        _sc_diff_kernel(xw, src[k * es:(k + 1) * es], dst[k * es:(k + 1) * es], es)
        for k in range(N_SLICES)
    ]
    acc = None
    for k in range(N_SLICES):
        diff = jax.lax.bitcast_convert_type(
            diffs[k], jnp.bfloat16).reshape(es, D_IN)
        acc = _tc_mlp_slice(diff, W1b, b1r, W2b, b2r, acc,
                            k * (es // BLOCK_E))
    return acc


# speedup vs baseline: 2.4559x; 2.4559x over previous
import functools

import jax
import jax.numpy as jnp
from jax import lax
from jax.experimental import pallas as pl
from jax.experimental.pallas import tpu as pltpu
from jax.experimental.pallas import tpu_sc as plsc

N_NODES = 10000
D_IN = 128
N_EDGES = 320000

NUM_CORES = 2
NUM_SUBCORES = 16
NUM_WORKERS = NUM_CORES * NUM_SUBCORES

CHUNK_R = 40
ROWS_PER_STEP = 4


def _sc_diff_kernel(x, src_lo, dst_lo, src_hi, dst_hi, half):
    mesh = plsc.VectorSubcoreMesh(
        core_axis_name="c", subcore_axis_name="s",
        num_cores=NUM_CORES, num_subcores=NUM_SUBCORES)
    rpw = half // NUM_WORKERS
    num_chunks = rpw // CHUNK_R
    assert half % NUM_WORKERS == 0 and rpw % CHUNK_R == 0 and num_chunks >= 2
    assert rpw % 8 == 0 and CHUNK_R % 8 == 0

    @functools.partial(
        pl.kernel,
        out_type=jax.ShapeDtypeStruct((half, D_IN), jnp.float32),
        mesh=mesh,
        compiler_params=pltpu.CompilerParams(needs_layout_passes=False),
        scratch_types=(
            [pltpu.VMEM((rpw,), jnp.int32)] * 4
            + [pltpu.VMEM((CHUNK_R, D_IN), jnp.float32)] * 8
            + [pltpu.SemaphoreType.DMA] * 4
        ),
    )
    def k(x_hbm, slo_hbm, dlo_hbm, shi_hbm, dhi_hbm, out_hbm,
          islo, idlo, ishi, idhi,
          rslo0, rslo1, rdlo0, rdlo1, rshi0, rshi1, rdhi0, rdhi1,
          sem_g0, sem_g1, sem_w0, sem_w1):
        wid = lax.axis_index("s") * NUM_CORES + lax.axis_index("c")
        base_r = wid * rpw
        rslo = (rslo0, rslo1)
        rdlo = (rdlo0, rdlo1)
        rshi = (rshi0, rshi1)
        rdhi = (rdhi0, rdhi1)
        sem_g = (sem_g0, sem_g1)
        sem_w = (sem_w0, sem_w1)

        def gather_descs(c, p):
            isl = pl.ds(c * CHUNK_R, CHUNK_R)
            return [
                (x_hbm.at[islo.at[isl]], rslo[p], sem_g[p]),
                (x_hbm.at[idlo.at[isl]], rdlo[p], sem_g[p]),
                (x_hbm.at[ishi.at[isl]], rshi[p], sem_g[p]),
                (x_hbm.at[idhi.at[isl]], rdhi[p], sem_g[p]),
            ]

        def when(pred, fn):
            if isinstance(pred, bool):
                if pred:
                    fn()
            else:
                pl.when(pred)(fn)

        def substep(c, p):
            pp = 1 - p

            def wb_wait():
                pltpu.make_async_copy(
                    rdlo[pp], out_hbm.at[pl.ds(base_r, CHUNK_R)], sem_w[pp]
                ).wait()

            def fire_next():
                for s_, d_, sm in gather_descs(c + 1, pp):
                    pltpu.async_copy(s_, d_, sm)

            when(c > 0, wb_wait)
            when(c + 1 < num_chunks, fire_next)

            for s_, d_, sm in gather_descs(c, p):
                pltpu.make_async_copy(s_, d_, sm).wait()

            def row_body(i, carry2):
                for rr in range(ROWS_PER_STEP):
                    r = i * ROWS_PER_STEP + rr
                    for kk in range(D_IN // 16):
                        s = pl.ds(kk * 16, 16)
                        lo = jnp.abs(rdlo[p][r, s] - rslo[p][r, s])
                        hi = jnp.abs(rdhi[p][r, s] - rshi[p][r, s])
                        lo_u = plsc.bitcast(lo, jnp.int32)
                        hi_u = plsc.bitcast(hi, jnp.int32)
                        w = lax.shift_right_logical(lo_u + 0x8000, 16) | (
                            (hi_u + 0x8000) & jnp.int32(-65536))
                        rdlo[p][r, s] = plsc.bitcast(w, jnp.float32)
                return carry2

            lax.fori_loop(0, CHUNK_R // ROWS_PER_STEP, row_body, 0)

            pltpu.async_copy(
                rdlo[p], out_hbm.at[pl.ds(base_r + c * CHUNK_R, CHUNK_R)],
                sem_w[p])

        pltpu.sync_copy(slo_hbm.at[pl.ds(base_r, rpw)], islo)
        pltpu.sync_copy(dlo_hbm.at[pl.ds(base_r, rpw)], idlo)
        pltpu.sync_copy(shi_hbm.at[pl.ds(base_r, rpw)], ishi)
        pltpu.sync_copy(dhi_hbm.at[pl.ds(base_r, rpw)], idhi)
        for s_, d_, sm in gather_descs(0, 0):
            pltpu.async_copy(s_, d_, sm)

        substep(0, 0)

        def pair_body(i, carry):
            substep(2 * i + 1, 1)
            substep(2 * i + 2, 0)
            return carry

        lax.fori_loop(0, (num_chunks - 1) // 2, pair_body, 0)
        if (num_chunks - 1) % 2 == 1:
            substep(num_chunks - 1, 1)

        last_p = (num_chunks - 1) % 2
        pltpu.make_async_copy(
            rdlo[last_p], out_hbm.at[pl.ds(base_r, CHUNK_R)], sem_w[last_p]
        ).wait()

    return k(x, src_lo, dst_lo, src_hi, dst_hi)


BLOCK_P = 1600
N_SLICES = 1


def _tc_mlp_compute(packed_ref, w1_ref, b1_ref, w2_ref, b2_ref, out_ref):
    kk = pl.program_id(1)
    bits = lax.bitcast_convert_type(packed_ref[...], jnp.int32)
    half_bits = jnp.where(kk == 0,
                          lax.shift_left(bits, 16),
                          bits & jnp.int32(-65536))
    d = lax.bitcast_convert_type(half_bits, jnp.float32).astype(jnp.bfloat16)
    h = jnp.dot(d, w1_ref[...], preferred_element_type=jnp.float32)
    h = jnp.maximum(h + b1_ref[...], 0.0)
    e = jnp.dot(h.astype(jnp.bfloat16), w2_ref[...],
                preferred_element_type=jnp.float32)
    out_ref[...] = jax.nn.sigmoid(e + b2_ref[...])


def _tc_mlp_body(packed_ref, w1_ref, b1_ref, w2_ref, b2_ref, acc_ref, out_ref):
    del acc_ref
    _tc_mlp_compute(packed_ref, w1_ref, b1_ref, w2_ref, b2_ref, out_ref)


_WEIGHT_SPECS = [
    pl.BlockSpec((D_IN, 64), lambda i, k: (0, 0)),
    pl.BlockSpec((1, 64), lambda i, k: (0, 0)),
    pl.BlockSpec((64, D_IN), lambda i, k: (0, 0)),
    pl.BlockSpec((1, D_IN), lambda i, k: (0, 0)),
]


def _tc_mlp_slice(packed, W1, b1, W2, b2, acc, block_base):
    nblk = packed.shape[0] // BLOCK_P
    grid = (nblk, 2)
    dspec = pl.BlockSpec((BLOCK_P, D_IN), lambda i, k: (i, 0))
    ospec = pl.BlockSpec(
        (BLOCK_P, D_IN), lambda i, k: (block_base + i + k * nblk, 0))
    oshape = jax.ShapeDtypeStruct((N_EDGES, D_IN), jnp.float32)
    if acc is None:
        return pl.pallas_call(
            _tc_mlp_compute, grid=grid,
            in_specs=[dspec] + _WEIGHT_SPECS,
            out_specs=ospec, out_shape=oshape,
        )(packed, W1, b1, W2, b2)
    return pl.pallas_call(
        _tc_mlp_body, grid=grid,
        in_specs=[dspec] + _WEIGHT_SPECS
        + [pl.BlockSpec(memory_space=pl.ANY)],
        out_specs=ospec, out_shape=oshape,
        input_output_aliases={5: 0},
    )(packed, W1, b1, W2, b2, acc)


def kernel(x, edge_index, W1, b1, W2, b2):
    src = edge_index[0]
    dst = edge_index[1]
    W1b = W1.astype(jnp.bfloat16)
    W2b = W2.astype(jnp.bfloat16)
    b1r = b1.reshape(1, 64)
    b2r = b2.reshape(1, 128)
    es = N_EDGES // N_SLICES
    half = es // 2

    packs = []
    for k in range(N_SLICES):
        s_k = src[k * es:(k + 1) * es]
        d_k = dst[k * es:(k + 1) * es]
        packs.append(_sc_diff_kernel(
            x, s_k[:half], d_k[:half], s_k[half:], d_k[half:], half))

    acc = None
    nblk = half // BLOCK_P
    for k in range(N_SLICES):
        acc = _tc_mlp_slice(packs[k], W1b, b1r, W2b, b2r, acc,
                            k * 2 * nblk)
    return acc


# speedup vs baseline: 2.7948x; 1.1380x over previous
import functools

import jax
import jax.numpy as jnp
from jax import lax
from jax.experimental import pallas as pl
from jax.experimental.pallas import tpu as pltpu
from jax.experimental.pallas import tpu_sc as plsc

N_NODES = 10000
D_IN = 128
N_EDGES = 320000

NUM_CORES = 2
NUM_SUBCORES = 16
NUM_WORKERS = NUM_CORES * NUM_SUBCORES

CHUNK_R = 40
ROWS_PER_STEP = 4


def _sc_diff_kernel(x, src_lo, dst_lo, src_hi, dst_hi, half):
    mesh = plsc.VectorSubcoreMesh(
        core_axis_name="c", subcore_axis_name="s",
        num_cores=NUM_CORES, num_subcores=NUM_SUBCORES)
    rpw = half // NUM_WORKERS
    num_chunks = rpw // CHUNK_R
    assert half % NUM_WORKERS == 0 and rpw % CHUNK_R == 0 and num_chunks >= 2
    assert rpw % 8 == 0 and CHUNK_R % 8 == 0

    @functools.partial(
        pl.kernel,
        out_type=jax.ShapeDtypeStruct((half, D_IN), jnp.float32),
        mesh=mesh,
        compiler_params=pltpu.CompilerParams(needs_layout_passes=False),
        scratch_types=(
            [pltpu.VMEM((rpw,), jnp.int32)] * 4
            + [pltpu.VMEM((CHUNK_R, D_IN), jnp.float32)] * 8
            + [pltpu.SemaphoreType.DMA] * 4
        ),
    )
    def k(x_hbm, slo_hbm, dlo_hbm, shi_hbm, dhi_hbm, out_hbm,
          islo, idlo, ishi, idhi,
          rslo0, rslo1, rdlo0, rdlo1, rshi0, rshi1, rdhi0, rdhi1,
          sem_g0, sem_g1, sem_w0, sem_w1):
        wid = lax.axis_index("s") * NUM_CORES + lax.axis_index("c")
        base_r = wid * rpw
        rslo = (rslo0, rslo1)
        rdlo = (rdlo0, rdlo1)
        rshi = (rshi0, rshi1)
        rdhi = (rdhi0, rdhi1)
        sem_g = (sem_g0, sem_g1)
        sem_w = (sem_w0, sem_w1)

        def gather_descs(c, p):
            isl = pl.ds(c * CHUNK_R, CHUNK_R)
            return [
                (x_hbm.at[islo.at[isl]], rslo[p], sem_g[p]),
                (x_hbm.at[idlo.at[isl]], rdlo[p], sem_g[p]),
                (x_hbm.at[ishi.at[isl]], rshi[p], sem_g[p]),
                (x_hbm.at[idhi.at[isl]], rdhi[p], sem_g[p]),
            ]

        def when(pred, fn):
            if isinstance(pred, bool):
                if pred:
                    fn()
            else:
                pl.when(pred)(fn)

        def substep(c, p):
            pp = 1 - p

            def wb_wait():
                pltpu.make_async_copy(
                    rdlo[pp], out_hbm.at[pl.ds(base_r, CHUNK_R)], sem_w[pp]
                ).wait()

            def fire_next():
                for s_, d_, sm in gather_descs(c + 1, pp):
                    pltpu.async_copy(s_, d_, sm)

            when(c > 0, wb_wait)
            when(c + 1 < num_chunks, fire_next)

            for s_, d_, sm in gather_descs(c, p):
                pltpu.make_async_copy(s_, d_, sm).wait()

            def row_body(i, carry2):
                for rr in range(ROWS_PER_STEP):
                    r = i * ROWS_PER_STEP + rr
                    for kk in range(D_IN // 16):
                        s = pl.ds(kk * 16, 16)
                        lo = jnp.abs(rdlo[p][r, s] - rslo[p][r, s])
                        hi = jnp.abs(rdhi[p][r, s] - rshi[p][r, s])
                        lo_u = plsc.bitcast(lo, jnp.int32)
                        hi_u = plsc.bitcast(hi, jnp.int32)
                        w = lax.shift_right_logical(lo_u + 0x8000, 16) | (
                            (hi_u + 0x8000) & jnp.int32(-65536))
                        rdlo[p][r, s] = plsc.bitcast(w, jnp.float32)
                return carry2

            lax.fori_loop(0, CHUNK_R // ROWS_PER_STEP, row_body, 0)

            pltpu.async_copy(
                rdlo[p], out_hbm.at[pl.ds(base_r + c * CHUNK_R, CHUNK_R)],
                sem_w[p])

        pltpu.sync_copy(slo_hbm.at[pl.ds(base_r, rpw)], islo)
        pltpu.sync_copy(dlo_hbm.at[pl.ds(base_r, rpw)], idlo)
        pltpu.sync_copy(shi_hbm.at[pl.ds(base_r, rpw)], ishi)
        pltpu.sync_copy(dhi_hbm.at[pl.ds(base_r, rpw)], idhi)
        for s_, d_, sm in gather_descs(0, 0):
            pltpu.async_copy(s_, d_, sm)

        substep(0, 0)

        def pair_body(i, carry):
            substep(2 * i + 1, 1)
            substep(2 * i + 2, 0)
            return carry

        lax.fori_loop(0, (num_chunks - 1) // 2, pair_body, 0)
        if (num_chunks - 1) % 2 == 1:
            substep(num_chunks - 1, 1)

        last_p = (num_chunks - 1) % 2
        pltpu.make_async_copy(
            rdlo[last_p], out_hbm.at[pl.ds(base_r, CHUNK_R)], sem_w[last_p]
        ).wait()

    return k(x, src_lo, dst_lo, src_hi, dst_hi)


BLOCK_P = 3200
N_SLICES = 1


def _tc_mlp_compute(packed_ref, w1_ref, b1_ref, w2_ref, b2_ref, out_ref):
    kk = pl.program_id(1)
    bits = lax.bitcast_convert_type(packed_ref[...], jnp.int32)
    half_bits = jnp.where(kk == 0,
                          lax.shift_left(bits, 16),
                          bits & jnp.int32(-65536))
    d = lax.bitcast_convert_type(half_bits, jnp.float32).astype(jnp.bfloat16)
    h = jnp.dot(d, w1_ref[...], preferred_element_type=jnp.float32)
    h = jnp.maximum(h + b1_ref[...], 0.0)
    e = jnp.dot(h.astype(jnp.bfloat16), w2_ref[...],
                preferred_element_type=jnp.float32)
    out_ref[...] = jax.nn.sigmoid(e + b2_ref[...])


def _tc_mlp_body(packed_ref, w1_ref, b1_ref, w2_ref, b2_ref, acc_ref, out_ref):
    del acc_ref
    _tc_mlp_compute(packed_ref, w1_ref, b1_ref, w2_ref, b2_ref, out_ref)


_WEIGHT_SPECS = [
    pl.BlockSpec((D_IN, 64), lambda i, k: (0, 0)),
    pl.BlockSpec((1, 64), lambda i, k: (0, 0)),
    pl.BlockSpec((64, D_IN), lambda i, k: (0, 0)),
    pl.BlockSpec((1, D_IN), lambda i, k: (0, 0)),
]


def _tc_mlp_slice(packed, W1, b1, W2, b2, acc, block_base):
    nblk = packed.shape[0] // BLOCK_P
    grid = (nblk, 2)
    dspec = pl.BlockSpec((BLOCK_P, D_IN), lambda i, k: (i, 0))
    ospec = pl.BlockSpec(
        (BLOCK_P, D_IN), lambda i, k: (block_base + i + k * nblk, 0))
    oshape = jax.ShapeDtypeStruct((N_EDGES, D_IN), jnp.float32)
    if acc is None:
        return pl.pallas_call(
            _tc_mlp_compute, grid=grid,
            in_specs=[dspec] + _WEIGHT_SPECS,
            out_specs=ospec, out_shape=oshape,
        )(packed, W1, b1, W2, b2)
    return pl.pallas_call(
        _tc_mlp_body, grid=grid,
        in_specs=[dspec] + _WEIGHT_SPECS
        + [pl.BlockSpec(memory_space=pl.ANY)],
        out_specs=ospec, out_shape=oshape,
        input_output_aliases={5: 0},
    )(packed, W1, b1, W2, b2, acc)


def kernel(x, edge_index, W1, b1, W2, b2):
    src = edge_index[0]
    dst = edge_index[1]
    W1b = W1.astype(jnp.bfloat16)
    W2b = W2.astype(jnp.bfloat16)
    b1r = b1.reshape(1, 64)
    b2r = b2.reshape(1, 128)
    es = N_EDGES // N_SLICES
    half = es // 2

    packs = []
    for k in range(N_SLICES):
        s_k = src[k * es:(k + 1) * es]
        d_k = dst[k * es:(k + 1) * es]
        packs.append(_sc_diff_kernel(
            x, s_k[:half], d_k[:half], s_k[half:], d_k[half:], half))

    acc = None
    nblk = half // BLOCK_P
    for k in range(N_SLICES):
        acc = _tc_mlp_slice(packs[k], W1b, b1r, W2b, b2r, acc,
                            k * 2 * nblk)
    return acc


# speedup vs baseline: 3.0487x; 1.0909x over previous
import functools

import jax
import jax.numpy as jnp
from jax import lax
from jax.experimental import pallas as pl
from jax.experimental.pallas import tpu as pltpu
from jax.experimental.pallas import tpu_sc as plsc

N_NODES = 10000
D_IN = 128
N_EDGES = 320000

NUM_CORES = 2
NUM_SUBCORES = 16
NUM_WORKERS = NUM_CORES * NUM_SUBCORES

CHUNK_R = 40
ROWS_PER_STEP = 4


def _sc_diff_kernel(x, src_lo, dst_lo, src_hi, dst_hi, half):
    mesh = plsc.VectorSubcoreMesh(
        core_axis_name="c", subcore_axis_name="s",
        num_cores=NUM_CORES, num_subcores=NUM_SUBCORES)
    rpw = half // NUM_WORKERS
    num_chunks = rpw // CHUNK_R
    assert half % NUM_WORKERS == 0 and rpw % CHUNK_R == 0 and num_chunks >= 2
    assert rpw % 8 == 0 and CHUNK_R % 8 == 0

    @functools.partial(
        pl.kernel,
        out_type=jax.ShapeDtypeStruct((half, D_IN), jnp.float32),
        mesh=mesh,
        compiler_params=pltpu.CompilerParams(needs_layout_passes=False),
        scratch_types=(
            [pltpu.VMEM((rpw,), jnp.int32)] * 4
            + [pltpu.VMEM((CHUNK_R, D_IN), jnp.float32)] * 8
            + [pltpu.SemaphoreType.DMA] * 4
        ),
    )
    def k(x_hbm, slo_hbm, dlo_hbm, shi_hbm, dhi_hbm, out_hbm,
          islo, idlo, ishi, idhi,
          rslo0, rslo1, rdlo0, rdlo1, rshi0, rshi1, rdhi0, rdhi1,
          sem_g0, sem_g1, sem_w0, sem_w1):
        wid = lax.axis_index("s") * NUM_CORES + lax.axis_index("c")
        base_r = wid * rpw
        rslo = (rslo0, rslo1)
        rdlo = (rdlo0, rdlo1)
        rshi = (rshi0, rshi1)
        rdhi = (rdhi0, rdhi1)
        sem_g = (sem_g0, sem_g1)
        sem_w = (sem_w0, sem_w1)

        def gather_descs(c, p):
            isl = pl.ds(c * CHUNK_R, CHUNK_R)
            return [
                (x_hbm.at[islo.at[isl]], rslo[p], sem_g[p]),
                (x_hbm.at[idlo.at[isl]], rdlo[p], sem_g[p]),
                (x_hbm.at[ishi.at[isl]], rshi[p], sem_g[p]),
                (x_hbm.at[idhi.at[isl]], rdhi[p], sem_g[p]),
            ]

        def when(pred, fn):
            if isinstance(pred, bool):
                if pred:
                    fn()
            else:
                pl.when(pred)(fn)

        def substep(c, p):
            pp = 1 - p

            def wb_wait():
                pltpu.make_async_copy(
                    rdlo[pp], out_hbm.at[pl.ds(base_r, CHUNK_R)], sem_w[pp]
                ).wait()

            def fire_next():
                for s_, d_, sm in gather_descs(c + 1, pp):
                    pltpu.async_copy(s_, d_, sm)

            when(c > 0, wb_wait)
            when(c + 1 < num_chunks, fire_next)

            for s_, d_, sm in gather_descs(c, p):
                pltpu.make_async_copy(s_, d_, sm).wait()

            def row_body(i, carry2):
                for rr in range(ROWS_PER_STEP):
                    r = i * ROWS_PER_STEP + rr
                    for kk in range(D_IN // 16):
                        s = pl.ds(kk * 16, 16)
                        lo = jnp.abs(rdlo[p][r, s] - rslo[p][r, s])
                        hi = jnp.abs(rdhi[p][r, s] - rshi[p][r, s])
                        lo_u = plsc.bitcast(lo, jnp.int32)
                        hi_u = plsc.bitcast(hi, jnp.int32)
                        w = lax.shift_right_logical(lo_u + 0x8000, 16) | (
                            (hi_u + 0x8000) & jnp.int32(-65536))
                        rdlo[p][r, s] = plsc.bitcast(w, jnp.float32)
                return carry2

            lax.fori_loop(0, CHUNK_R // ROWS_PER_STEP, row_body, 0)

            pltpu.async_copy(
                rdlo[p], out_hbm.at[pl.ds(base_r + c * CHUNK_R, CHUNK_R)],
                sem_w[p])

        pltpu.sync_copy(slo_hbm.at[pl.ds(base_r, rpw)], islo)
        pltpu.sync_copy(dlo_hbm.at[pl.ds(base_r, rpw)], idlo)
        pltpu.sync_copy(shi_hbm.at[pl.ds(base_r, rpw)], ishi)
        pltpu.sync_copy(dhi_hbm.at[pl.ds(base_r, rpw)], idhi)
        for s_, d_, sm in gather_descs(0, 0):
            pltpu.async_copy(s_, d_, sm)

        substep(0, 0)

        def pair_body(i, carry):
            substep(2 * i + 1, 1)
            substep(2 * i + 2, 0)
            return carry

        lax.fori_loop(0, (num_chunks - 1) // 2, pair_body, 0)
        if (num_chunks - 1) % 2 == 1:
            substep(num_chunks - 1, 1)

        last_p = (num_chunks - 1) % 2
        pltpu.make_async_copy(
            rdlo[last_p], out_hbm.at[pl.ds(base_r, CHUNK_R)], sem_w[last_p]
        ).wait()

    return k(x, src_lo, dst_lo, src_hi, dst_hi)


BLOCK_P = 6400
N_SLICES = 1


def _tc_mlp_compute(packed_ref, w1_ref, b1_ref, w2_ref, b2_ref, out_ref):
    kk = pl.program_id(1)
    bits = lax.bitcast_convert_type(packed_ref[...], jnp.int32)
    half_bits = jnp.where(kk == 0,
                          lax.shift_left(bits, 16),
                          bits & jnp.int32(-65536))
    d = lax.bitcast_convert_type(half_bits, jnp.float32).astype(jnp.bfloat16)
    h = jnp.dot(d, w1_ref[...], preferred_element_type=jnp.float32)
    h = jnp.maximum(h + b1_ref[...], 0.0)
    e = jnp.dot(h.astype(jnp.bfloat16), w2_ref[...],
                preferred_element_type=jnp.float32)
    out_ref[...] = jax.nn.sigmoid(e + b2_ref[...])


def _tc_mlp_body(packed_ref, w1_ref, b1_ref, w2_ref, b2_ref, acc_ref, out_ref):
    del acc_ref
    _tc_mlp_compute(packed_ref, w1_ref, b1_ref, w2_ref, b2_ref, out_ref)


_WEIGHT_SPECS = [
    pl.BlockSpec((D_IN, 64), lambda i, k: (0, 0)),
    pl.BlockSpec((1, 64), lambda i, k: (0, 0)),
    pl.BlockSpec((64, D_IN), lambda i, k: (0, 0)),
    pl.BlockSpec((1, D_IN), lambda i, k: (0, 0)),
]


def _tc_mlp_slice(packed, W1, b1, W2, b2, acc, block_base):
    nblk = packed.shape[0] // BLOCK_P
    grid = (nblk, 2)
    dspec = pl.BlockSpec((BLOCK_P, D_IN), lambda i, k: (i, 0))
    ospec = pl.BlockSpec(
        (BLOCK_P, D_IN), lambda i, k: (block_base + i + k * nblk, 0))
    oshape = jax.ShapeDtypeStruct((N_EDGES, D_IN), jnp.float32)
    if acc is None:
        return pl.pallas_call(
            _tc_mlp_compute, grid=grid,
            in_specs=[dspec] + _WEIGHT_SPECS,
            out_specs=ospec, out_shape=oshape,
        )(packed, W1, b1, W2, b2)
    return pl.pallas_call(
        _tc_mlp_body, grid=grid,
        in_specs=[dspec] + _WEIGHT_SPECS
        + [pl.BlockSpec(memory_space=pl.ANY)],
        out_specs=ospec, out_shape=oshape,
        input_output_aliases={5: 0},
    )(packed, W1, b1, W2, b2, acc)


def kernel(x, edge_index, W1, b1, W2, b2):
    src = edge_index[0]
    dst = edge_index[1]
    W1b = W1.astype(jnp.bfloat16)
    W2b = W2.astype(jnp.bfloat16)
    b1r = b1.reshape(1, 64)
    b2r = b2.reshape(1, 128)
    es = N_EDGES // N_SLICES
    half = es // 2

    packs = []
    for k in range(N_SLICES):
        s_k = src[k * es:(k + 1) * es]
        d_k = dst[k * es:(k + 1) * es]
        packs.append(_sc_diff_kernel(
            x, s_k[:half], d_k[:half], s_k[half:], d_k[half:], half))

    acc = None
    nblk = half // BLOCK_P
    for k in range(N_SLICES):
        acc = _tc_mlp_slice(packs[k], W1b, b1r, W2b, b2r, acc,
                            k * 2 * nblk)
    return acc


# speedup vs baseline: 3.1167x; 1.0223x over previous
import functools

import jax
import jax.numpy as jnp
from jax import lax
from jax.experimental import pallas as pl
from jax.experimental.pallas import tpu as pltpu
from jax.experimental.pallas import tpu_sc as plsc

N_NODES = 10000
D_IN = 128
N_EDGES = 320000

NUM_CORES = 2
NUM_SUBCORES = 16
NUM_WORKERS = NUM_CORES * NUM_SUBCORES

CHUNK_R = 40
ROWS_PER_STEP = 4


def _sc_diff_kernel(x, src_lo, dst_lo, src_hi, dst_hi, half):
    mesh = plsc.VectorSubcoreMesh(
        core_axis_name="c", subcore_axis_name="s",
        num_cores=NUM_CORES, num_subcores=NUM_SUBCORES)
    rpw = half // NUM_WORKERS
    num_chunks = rpw // CHUNK_R
    assert half % NUM_WORKERS == 0 and rpw % CHUNK_R == 0 and num_chunks >= 2
    assert rpw % 8 == 0 and CHUNK_R % 8 == 0

    @functools.partial(
        pl.kernel,
        out_type=jax.ShapeDtypeStruct((half, D_IN), jnp.float32),
        mesh=mesh,
        compiler_params=pltpu.CompilerParams(needs_layout_passes=False),
        scratch_types=(
            [pltpu.VMEM((rpw,), jnp.int32)] * 4
            + [pltpu.VMEM((CHUNK_R, D_IN), jnp.float32)] * 8
            + [pltpu.SemaphoreType.DMA] * 4
        ),
    )
    def k(x_hbm, slo_hbm, dlo_hbm, shi_hbm, dhi_hbm, out_hbm,
          islo, idlo, ishi, idhi,
          rslo0, rslo1, rdlo0, rdlo1, rshi0, rshi1, rdhi0, rdhi1,
          sem_g0, sem_g1, sem_w0, sem_w1):
        wid = lax.axis_index("s") * NUM_CORES + lax.axis_index("c")
        base_r = wid * rpw
        rslo = (rslo0, rslo1)
        rdlo = (rdlo0, rdlo1)
        rshi = (rshi0, rshi1)
        rdhi = (rdhi0, rdhi1)
        sem_g = (sem_g0, sem_g1)
        sem_w = (sem_w0, sem_w1)

        def gather_descs(c, p):
            isl = pl.ds(c * CHUNK_R, CHUNK_R)
            return [
                (x_hbm.at[islo.at[isl]], rslo[p], sem_g[p]),
                (x_hbm.at[idlo.at[isl]], rdlo[p], sem_g[p]),
                (x_hbm.at[ishi.at[isl]], rshi[p], sem_g[p]),
                (x_hbm.at[idhi.at[isl]], rdhi[p], sem_g[p]),
            ]

        def when(pred, fn):
            if isinstance(pred, bool):
                if pred:
                    fn()
            else:
                pl.when(pred)(fn)

        def substep(c, p):
            pp = 1 - p

            def wb_wait():
                pltpu.make_async_copy(
                    rdlo[pp], out_hbm.at[pl.ds(base_r, CHUNK_R)], sem_w[pp]
                ).wait()

            def fire_next():
                for s_, d_, sm in gather_descs(c + 1, pp):
                    pltpu.async_copy(s_, d_, sm)

            when(c > 0, wb_wait)
            when(c + 1 < num_chunks, fire_next)

            for s_, d_, sm in gather_descs(c, p):
                pltpu.make_async_copy(s_, d_, sm).wait()

            def row_body(i, carry2):
                for rr in range(ROWS_PER_STEP):
                    r = i * ROWS_PER_STEP + rr
                    for kk in range(D_IN // 16):
                        s = pl.ds(kk * 16, 16)
                        lo = jnp.abs(rdlo[p][r, s] - rslo[p][r, s])
                        hi = jnp.abs(rdhi[p][r, s] - rshi[p][r, s])
                        lo_u = plsc.bitcast(lo, jnp.int32)
                        hi_u = plsc.bitcast(hi, jnp.int32)
                        w = lax.shift_right_logical(lo_u + 0x8000, 16) | (
                            (hi_u + 0x8000) & jnp.int32(-65536))
                        rdlo[p][r, s] = plsc.bitcast(w, jnp.float32)
                return carry2

            lax.fori_loop(0, CHUNK_R // ROWS_PER_STEP, row_body, 0)

            pltpu.async_copy(
                rdlo[p], out_hbm.at[pl.ds(base_r + c * CHUNK_R, CHUNK_R)],
                sem_w[p])

        pltpu.sync_copy(slo_hbm.at[pl.ds(base_r, rpw)], islo)
        pltpu.sync_copy(dlo_hbm.at[pl.ds(base_r, rpw)], idlo)
        pltpu.sync_copy(shi_hbm.at[pl.ds(base_r, rpw)], ishi)
        pltpu.sync_copy(dhi_hbm.at[pl.ds(base_r, rpw)], idhi)
        for s_, d_, sm in gather_descs(0, 0):
            pltpu.async_copy(s_, d_, sm)

        substep(0, 0)

        def pair_body(i, carry):
            substep(2 * i + 1, 1)
            substep(2 * i + 2, 0)
            return carry

        lax.fori_loop(0, (num_chunks - 1) // 2, pair_body, 0)
        if (num_chunks - 1) % 2 == 1:
            substep(num_chunks - 1, 1)

        last_p = (num_chunks - 1) % 2
        pltpu.make_async_copy(
            rdlo[last_p], out_hbm.at[pl.ds(base_r, CHUNK_R)], sem_w[last_p]
        ).wait()

    return k(x, src_lo, dst_lo, src_hi, dst_hi)


BLOCK_P = 8000
N_SLICES = 1


def _tc_mlp_compute(packed_ref, w1_ref, b1_ref, w2_ref, b2_ref, out_ref):
    kk = pl.program_id(1)
    bits = lax.bitcast_convert_type(packed_ref[...], jnp.int32)
    half_bits = jnp.where(kk == 0,
                          lax.shift_left(bits, 16),
                          bits & jnp.int32(-65536))
    d = lax.bitcast_convert_type(half_bits, jnp.float32).astype(jnp.bfloat16)
    h = jnp.dot(d, w1_ref[...], preferred_element_type=jnp.float32)
    h = jnp.maximum(h + b1_ref[...], 0.0)
    e = jnp.dot(h.astype(jnp.bfloat16), w2_ref[...],
                preferred_element_type=jnp.float32)
    out_ref[...] = jax.nn.sigmoid(e + b2_ref[...])


def _tc_mlp_body(packed_ref, w1_ref, b1_ref, w2_ref, b2_ref, acc_ref, out_ref):
    del acc_ref
    _tc_mlp_compute(packed_ref, w1_ref, b1_ref, w2_ref, b2_ref, out_ref)


_WEIGHT_SPECS = [
    pl.BlockSpec((D_IN, 64), lambda i, k: (0, 0)),
    pl.BlockSpec((1, 64), lambda i, k: (0, 0)),
    pl.BlockSpec((64, D_IN), lambda i, k: (0, 0)),
    pl.BlockSpec((1, D_IN), lambda i, k: (0, 0)),
]


def _tc_mlp_slice(packed, W1, b1, W2, b2, acc, block_base):
    nblk = packed.shape[0] // BLOCK_P
    grid = (nblk, 2)
    dspec = pl.BlockSpec((BLOCK_P, D_IN), lambda i, k: (i, 0))
    ospec = pl.BlockSpec(
        (BLOCK_P, D_IN), lambda i, k: (block_base + i + k * nblk, 0))
    oshape = jax.ShapeDtypeStruct((N_EDGES, D_IN), jnp.float32)
    if acc is None:
        return pl.pallas_call(
            _tc_mlp_compute, grid=grid,
            in_specs=[dspec] + _WEIGHT_SPECS,
            out_specs=ospec, out_shape=oshape,
        )(packed, W1, b1, W2, b2)
    return pl.pallas_call(
        _tc_mlp_body, grid=grid,
        in_specs=[dspec] + _WEIGHT_SPECS
        + [pl.BlockSpec(memory_space=pl.ANY)],
        out_specs=ospec, out_shape=oshape,
        input_output_aliases={5: 0},
    )(packed, W1, b1, W2, b2, acc)


def kernel(x, edge_index, W1, b1, W2, b2):
    src = edge_index[0]
    dst = edge_index[1]
    W1b = W1.astype(jnp.bfloat16)
    W2b = W2.astype(jnp.bfloat16)
    b1r = b1.reshape(1, 64)
    b2r = b2.reshape(1, 128)
    es = N_EDGES // N_SLICES
    half = es // 2

    packs = []
    for k in range(N_SLICES):
        s_k = src[k * es:(k + 1) * es]
        d_k = dst[k * es:(k + 1) * es]
        packs.append(_sc_diff_kernel(
            x, s_k[:half], d_k[:half], s_k[half:], d_k[half:], half))

    acc = None
    nblk = half // BLOCK_P
    for k in range(N_SLICES):
        acc = _tc_mlp_slice(packs[k], W1b, b1r, W2b, b2r, acc,
                            k * 2 * nblk)
    return acc


# speedup vs baseline: 3.3309x; 1.0687x over previous
import functools

import jax
import jax.numpy as jnp
from jax import lax
from jax.experimental import pallas as pl
from jax.experimental.pallas import tpu as pltpu
from jax.experimental.pallas import tpu_sc as plsc

N_NODES = 10000
D_IN = 128
N_EDGES = 320000

NUM_CORES = 2
NUM_SUBCORES = 16
NUM_WORKERS = NUM_CORES * NUM_SUBCORES

CHUNK_R = 40
ROWS_PER_STEP = 4


def _sc_diff_kernel(x, src_lo, dst_lo, src_hi, dst_hi, half):
    mesh = plsc.VectorSubcoreMesh(
        core_axis_name="c", subcore_axis_name="s",
        num_cores=NUM_CORES, num_subcores=NUM_SUBCORES)
    rpw = half // NUM_WORKERS
    num_chunks = rpw // CHUNK_R
    assert half % NUM_WORKERS == 0 and rpw % CHUNK_R == 0 and num_chunks >= 2
    assert rpw % 8 == 0 and CHUNK_R % 8 == 0

    @functools.partial(
        pl.kernel,
        out_type=jax.ShapeDtypeStruct((half, D_IN), jnp.float32),
        mesh=mesh,
        compiler_params=pltpu.CompilerParams(needs_layout_passes=False),
        scratch_types=(
            [pltpu.VMEM((rpw,), jnp.int32)] * 4
            + [pltpu.VMEM((CHUNK_R, D_IN), jnp.float32)] * 8
            + [pltpu.SemaphoreType.DMA] * 4
        ),
    )
    def k(x_hbm, slo_hbm, dlo_hbm, shi_hbm, dhi_hbm, out_hbm,
          islo, idlo, ishi, idhi,
          rslo0, rslo1, rdlo0, rdlo1, rshi0, rshi1, rdhi0, rdhi1,
          sem_g0, sem_g1, sem_w0, sem_w1):
        wid = lax.axis_index("s") * NUM_CORES + lax.axis_index("c")
        base_r = wid * rpw
        rslo = (rslo0, rslo1)
        rdlo = (rdlo0, rdlo1)
        rshi = (rshi0, rshi1)
        rdhi = (rdhi0, rdhi1)
        sem_g = (sem_g0, sem_g1)
        sem_w = (sem_w0, sem_w1)

        def gather_descs(c, p):
            isl = pl.ds(c * CHUNK_R, CHUNK_R)
            return [
                (x_hbm.at[islo.at[isl]], rslo[p], sem_g[p]),
                (x_hbm.at[idlo.at[isl]], rdlo[p], sem_g[p]),
                (x_hbm.at[ishi.at[isl]], rshi[p], sem_g[p]),
                (x_hbm.at[idhi.at[isl]], rdhi[p], sem_g[p]),
            ]

        def when(pred, fn):
            if isinstance(pred, bool):
                if pred:
                    fn()
            else:
                pl.when(pred)(fn)

        def substep(c, p):
            pp = 1 - p

            def wb_wait():
                pltpu.make_async_copy(
                    rdlo[pp], out_hbm.at[pl.ds(base_r, CHUNK_R)], sem_w[pp]
                ).wait()

            def fire_next():
                for s_, d_, sm in gather_descs(c + 1, pp):
                    pltpu.async_copy(s_, d_, sm)

            when(c > 0, wb_wait)
            when(c + 1 < num_chunks, fire_next)

            for s_, d_, sm in gather_descs(c, p):
                pltpu.make_async_copy(s_, d_, sm).wait()

            def row_body(i, carry2):
                for rr in range(ROWS_PER_STEP):
                    r = i * ROWS_PER_STEP + rr
                    for kk in range(D_IN // 16):
                        s = pl.ds(kk * 16, 16)
                        lo = jnp.abs(rdlo[p][r, s] - rslo[p][r, s])
                        hi = jnp.abs(rdhi[p][r, s] - rshi[p][r, s])
                        lo_u = plsc.bitcast(lo, jnp.int32)
                        hi_u = plsc.bitcast(hi, jnp.int32)
                        w = lax.shift_right_logical(lo_u + 0x8000, 16) | (
                            (hi_u + 0x8000) & jnp.int32(-65536))
                        rdlo[p][r, s] = plsc.bitcast(w, jnp.float32)
                return carry2

            lax.fori_loop(0, CHUNK_R // ROWS_PER_STEP, row_body, 0)

            pltpu.async_copy(
                rdlo[p], out_hbm.at[pl.ds(base_r + c * CHUNK_R, CHUNK_R)],
                sem_w[p])

        pltpu.sync_copy(slo_hbm.at[pl.ds(base_r, rpw)], islo)
        pltpu.sync_copy(dlo_hbm.at[pl.ds(base_r, rpw)], idlo)
        pltpu.sync_copy(shi_hbm.at[pl.ds(base_r, rpw)], ishi)
        pltpu.sync_copy(dhi_hbm.at[pl.ds(base_r, rpw)], idhi)
        for s_, d_, sm in gather_descs(0, 0):
            pltpu.async_copy(s_, d_, sm)

        substep(0, 0)

        def pair_body(i, carry):
            substep(2 * i + 1, 1)
            substep(2 * i + 2, 0)
            return carry

        lax.fori_loop(0, (num_chunks - 1) // 2, pair_body, 0)
        if (num_chunks - 1) % 2 == 1:
            substep(num_chunks - 1, 1)

        last_p = (num_chunks - 1) % 2
        pltpu.make_async_copy(
            rdlo[last_p], out_hbm.at[pl.ds(base_r, CHUNK_R)], sem_w[last_p]
        ).wait()

    return k(x, src_lo, dst_lo, src_hi, dst_hi)


BLOCK_P = 16000
N_SLICES = 1


def _tc_mlp_compute(packed_ref, w1_ref, b1_ref, w2_ref, b2_ref, out_ref):
    kk = pl.program_id(1)
    bits = lax.bitcast_convert_type(packed_ref[...], jnp.int32)
    half_bits = jnp.where(kk == 0,
                          lax.shift_left(bits, 16),
                          bits & jnp.int32(-65536))
    d = lax.bitcast_convert_type(half_bits, jnp.float32).astype(jnp.bfloat16)
    h = jnp.dot(d, w1_ref[...], preferred_element_type=jnp.float32)
    h = jnp.maximum(h + b1_ref[...], 0.0)
    e = jnp.dot(h.astype(jnp.bfloat16), w2_ref[...],
                preferred_element_type=jnp.float32)
    out_ref[...] = jax.nn.sigmoid(e + b2_ref[...])


def _tc_mlp_body(packed_ref, w1_ref, b1_ref, w2_ref, b2_ref, acc_ref, out_ref):
    del acc_ref
    _tc_mlp_compute(packed_ref, w1_ref, b1_ref, w2_ref, b2_ref, out_ref)


_WEIGHT_SPECS = [
    pl.BlockSpec((D_IN, 64), lambda i, k: (0, 0)),
    pl.BlockSpec((1, 64), lambda i, k: (0, 0)),
    pl.BlockSpec((64, D_IN), lambda i, k: (0, 0)),
    pl.BlockSpec((1, D_IN), lambda i, k: (0, 0)),
]


def _tc_mlp_slice(packed, W1, b1, W2, b2, acc, block_base):
    nblk = packed.shape[0] // BLOCK_P
    grid = (nblk, 2)
    dspec = pl.BlockSpec((BLOCK_P, D_IN), lambda i, k: (i, 0))
    ospec = pl.BlockSpec(
        (BLOCK_P, D_IN), lambda i, k: (block_base + i + k * nblk, 0))
    oshape = jax.ShapeDtypeStruct((N_EDGES, D_IN), jnp.float32)
    if acc is None:
        return pl.pallas_call(
            _tc_mlp_compute, grid=grid,
            in_specs=[dspec] + _WEIGHT_SPECS,
            out_specs=ospec, out_shape=oshape,
        )(packed, W1, b1, W2, b2)
    return pl.pallas_call(
        _tc_mlp_body, grid=grid,
        in_specs=[dspec] + _WEIGHT_SPECS
        + [pl.BlockSpec(memory_space=pl.ANY)],
        out_specs=ospec, out_shape=oshape,
        input_output_aliases={5: 0},
    )(packed, W1, b1, W2, b2, acc)


def kernel(x, edge_index, W1, b1, W2, b2):
    src = edge_index[0]
    dst = edge_index[1]
    W1b = W1.astype(jnp.bfloat16)
    W2b = W2.astype(jnp.bfloat16)
    b1r = b1.reshape(1, 64)
    b2r = b2.reshape(1, 128)
    es = N_EDGES // N_SLICES
    half = es // 2

    packs = []
    for k in range(N_SLICES):
        s_k = src[k * es:(k + 1) * es]
        d_k = dst[k * es:(k + 1) * es]
        packs.append(_sc_diff_kernel(
            x, s_k[:half], d_k[:half], s_k[half:], d_k[half:], half))

    acc = None
    nblk = half // BLOCK_P
    for k in range(N_SLICES):
        acc = _tc_mlp_slice(packs[k], W1b, b1r, W2b, b2r, acc,
                            k * 2 * nblk)
    return acc


# speedup vs baseline: 3.3580x; 1.0081x over previous
import functools

import jax
import jax.numpy as jnp
from jax import lax
from jax.experimental import pallas as pl
from jax.experimental.pallas import tpu as pltpu
from jax.experimental.pallas import tpu_sc as plsc

N_NODES = 10000
D_IN = 128
N_EDGES = 320000

NUM_CORES = 2
NUM_SUBCORES = 16
NUM_WORKERS = NUM_CORES * NUM_SUBCORES

CHUNK_R = 40
ROWS_PER_STEP = 4


def _sc_diff_kernel(x, src_lo, dst_lo, src_hi, dst_hi, half):
    mesh = plsc.VectorSubcoreMesh(
        core_axis_name="c", subcore_axis_name="s",
        num_cores=NUM_CORES, num_subcores=NUM_SUBCORES)
    rpw = half // NUM_WORKERS
    num_chunks = rpw // CHUNK_R
    assert half % NUM_WORKERS == 0 and rpw % CHUNK_R == 0 and num_chunks >= 2
    assert rpw % 8 == 0 and CHUNK_R % 8 == 0

    @functools.partial(
        pl.kernel,
        out_type=jax.ShapeDtypeStruct((half, D_IN), jnp.float32),
        mesh=mesh,
        compiler_params=pltpu.CompilerParams(needs_layout_passes=False),
        scratch_types=(
            [pltpu.VMEM((rpw,), jnp.int32)] * 4
            + [pltpu.VMEM((CHUNK_R, D_IN), jnp.float32)] * 8
            + [pltpu.SemaphoreType.DMA] * 4
        ),
    )
    def k(x_hbm, slo_hbm, dlo_hbm, shi_hbm, dhi_hbm, out_hbm,
          islo, idlo, ishi, idhi,
          rslo0, rslo1, rdlo0, rdlo1, rshi0, rshi1, rdhi0, rdhi1,
          sem_g0, sem_g1, sem_w0, sem_w1):
        wid = lax.axis_index("s") * NUM_CORES + lax.axis_index("c")
        base_r = wid * rpw
        rslo = (rslo0, rslo1)
        rdlo = (rdlo0, rdlo1)
        rshi = (rshi0, rshi1)
        rdhi = (rdhi0, rdhi1)
        sem_g = (sem_g0, sem_g1)
        sem_w = (sem_w0, sem_w1)

        def gather_descs(c, p):
            isl = pl.ds(c * CHUNK_R, CHUNK_R)
            return [
                (x_hbm.at[islo.at[isl]], rslo[p], sem_g[p]),
                (x_hbm.at[idlo.at[isl]], rdlo[p], sem_g[p]),
                (x_hbm.at[ishi.at[isl]], rshi[p], sem_g[p]),
                (x_hbm.at[idhi.at[isl]], rdhi[p], sem_g[p]),
            ]

        def when(pred, fn):
            if isinstance(pred, bool):
                if pred:
                    fn()
            else:
                pl.when(pred)(fn)

        def substep(c, p):
            pp = 1 - p

            def wb_wait():
                pltpu.make_async_copy(
                    rdlo[pp], out_hbm.at[pl.ds(base_r, CHUNK_R)], sem_w[pp]
                ).wait()

            def fire_next():
                for s_, d_, sm in gather_descs(c + 1, pp):
                    pltpu.async_copy(s_, d_, sm)

            when(c > 0, wb_wait)
            when(c + 1 < num_chunks, fire_next)

            for s_, d_, sm in gather_descs(c, p):
                pltpu.make_async_copy(s_, d_, sm).wait()

            def row_body(i, carry2):
                for rr in range(ROWS_PER_STEP):
                    r = i * ROWS_PER_STEP + rr
                    for kk in range(D_IN // 16):
                        s = pl.ds(kk * 16, 16)
                        lo = jnp.abs(rdlo[p][r, s] - rslo[p][r, s])
                        hi = jnp.abs(rdhi[p][r, s] - rshi[p][r, s])
                        lo_u = plsc.bitcast(lo, jnp.int32)
                        hi_u = plsc.bitcast(hi, jnp.int32)
                        w = lax.shift_right_logical(lo_u + 0x8000, 16) | (
                            (hi_u + 0x8000) & jnp.int32(-65536))
                        rdlo[p][r, s] = plsc.bitcast(w, jnp.float32)
                return carry2

            lax.fori_loop(0, CHUNK_R // ROWS_PER_STEP, row_body, 0)

            pltpu.async_copy(
                rdlo[p], out_hbm.at[pl.ds(base_r + c * CHUNK_R, CHUNK_R)],
                sem_w[p])

        pltpu.sync_copy(slo_hbm.at[pl.ds(base_r, rpw)], islo)
        pltpu.sync_copy(dlo_hbm.at[pl.ds(base_r, rpw)], idlo)
        pltpu.sync_copy(shi_hbm.at[pl.ds(base_r, rpw)], ishi)
        pltpu.sync_copy(dhi_hbm.at[pl.ds(base_r, rpw)], idhi)
        for s_, d_, sm in gather_descs(0, 0):
            pltpu.async_copy(s_, d_, sm)

        substep(0, 0)

        def pair_body(i, carry):
            substep(2 * i + 1, 1)
            substep(2 * i + 2, 0)
            return carry

        lax.fori_loop(0, (num_chunks - 1) // 2, pair_body, 0)
        if (num_chunks - 1) % 2 == 1:
            substep(num_chunks - 1, 1)

        last_p = (num_chunks - 1) % 2
        pltpu.make_async_copy(
            rdlo[last_p], out_hbm.at[pl.ds(base_r, CHUNK_R)], sem_w[last_p]
        ).wait()

    return k(x, src_lo, dst_lo, src_hi, dst_hi)


BLOCK_P = 20000
N_SLICES = 1


def _tc_mlp_compute(packed_ref, w1_ref, b1_ref, w2_ref, b2_ref, out_ref):
    kk = pl.program_id(1)
    bits = lax.bitcast_convert_type(packed_ref[...], jnp.int32)
    half_bits = jnp.where(kk == 0,
                          lax.shift_left(bits, 16),
                          bits & jnp.int32(-65536))
    d = lax.bitcast_convert_type(half_bits, jnp.float32).astype(jnp.bfloat16)
    h = jnp.dot(d, w1_ref[...], preferred_element_type=jnp.float32)
    h = jnp.maximum(h + b1_ref[...], 0.0)
    e = jnp.dot(h.astype(jnp.bfloat16), w2_ref[...],
                preferred_element_type=jnp.float32)
    out_ref[...] = jax.nn.sigmoid(e + b2_ref[...])


def _tc_mlp_body(packed_ref, w1_ref, b1_ref, w2_ref, b2_ref, acc_ref, out_ref):
    del acc_ref
    _tc_mlp_compute(packed_ref, w1_ref, b1_ref, w2_ref, b2_ref, out_ref)


_WEIGHT_SPECS = [
    pl.BlockSpec((D_IN, 64), lambda i, k: (0, 0)),
    pl.BlockSpec((1, 64), lambda i, k: (0, 0)),
    pl.BlockSpec((64, D_IN), lambda i, k: (0, 0)),
    pl.BlockSpec((1, D_IN), lambda i, k: (0, 0)),
]


def _tc_mlp_slice(packed, W1, b1, W2, b2, acc, block_base):
    nblk = packed.shape[0] // BLOCK_P
    grid = (nblk, 2)
    dspec = pl.BlockSpec((BLOCK_P, D_IN), lambda i, k: (i, 0))
    ospec = pl.BlockSpec(
        (BLOCK_P, D_IN), lambda i, k: (block_base + i + k * nblk, 0))
    oshape = jax.ShapeDtypeStruct((N_EDGES, D_IN), jnp.float32)
    if acc is None:
        return pl.pallas_call(
            _tc_mlp_compute, grid=grid,
            in_specs=[dspec] + _WEIGHT_SPECS,
            out_specs=ospec, out_shape=oshape,
        )(packed, W1, b1, W2, b2)
    return pl.pallas_call(
        _tc_mlp_body, grid=grid,
        in_specs=[dspec] + _WEIGHT_SPECS
        + [pl.BlockSpec(memory_space=pl.ANY)],
        out_specs=ospec, out_shape=oshape,
        input_output_aliases={5: 0},
    )(packed, W1, b1, W2, b2, acc)


def kernel(x, edge_index, W1, b1, W2, b2):
    src = edge_index[0]
    dst = edge_index[1]
    W1b = W1.astype(jnp.bfloat16)
    W2b = W2.astype(jnp.bfloat16)
    b1r = b1.reshape(1, 64)
    b2r = b2.reshape(1, 128)
    es = N_EDGES // N_SLICES
    half = es // 2

    packs = []
    for k in range(N_SLICES):
        s_k = src[k * es:(k + 1) * es]
        d_k = dst[k * es:(k + 1) * es]
        packs.append(_sc_diff_kernel(
            x, s_k[:half], d_k[:half], s_k[half:], d_k[half:], half))

    acc = None
    nblk = half // BLOCK_P
    for k in range(N_SLICES):
        acc = _tc_mlp_slice(packs[k], W1b, b1r, W2b, b2r, acc,
                            k * 2 * nblk)
    return acc
